# Initial kernel scaffold; baseline (speedup 1.0000x reference)
import jax
import jax.numpy as jnp
from jax.experimental import pallas as pl


def kernel(x, edge_attr, edge_index, batch, W0, b0, W1, b1):
    raise NotImplementedError("write your pallas kernel here")


# speedup vs baseline: 19.8499x; 19.8499x over previous
import functools
import jax
import jax.numpy as jnp
from jax import lax
from jax.experimental import pallas as pl
from jax.experimental.pallas import tpu as pltpu
from jax.experimental.pallas import tpu_sc as plsc

E = 160000
N = 10000
IN = 16
HID = 64
G = 64
NC = 2
NS = 16
NW = NC * NS
NPT = 640
NPT_LAST = N - NPT * (NS - 1)

C_S = 1280
KS = C_S // 128
NCH_S = E // C_S
C_E = 256
KE = C_E // 128
NCH_E = E // C_E

_mesh = plsc.VectorSubcoreMesh(core_axis_name="c", subcore_axis_name="s")


def _wid():
    return lax.axis_index("s") * NC + lax.axis_index("c")


def _nchunks(wid, total):
    return (total - wid + (NW - 1)) // NW


def _zero_rows(ref, rows, width):
    def fill(i, _):
        for k in range(width // 16):
            ref[i, pl.ds(16 * k, 16)] = jnp.zeros((16,), jnp.float32)
        return 0
    lax.fori_loop(0, rows, fill, 0)


def _tile_slab(s):
    return NPT * s


def _copy_slab(src_v, dst, s):
    @pl.when(s < NS - 1)
    def _():
        pltpu.sync_copy(src_v.at[pl.ds(0, NPT)], dst.at[pl.ds(NPT * s, NPT)])

    @pl.when(s == NS - 1)
    def _():
        pltpu.sync_copy(src_v.at[pl.ds(0, NPT_LAST)],
                        dst.at[pl.ds(NPT * (NS - 1), NPT_LAST)])


def _copy_slab_out(sh, dst, c, s):
    @pl.when(s < NS - 1)
    def _():
        pltpu.sync_copy(sh.at[pl.ds(NPT * s, NPT)],
                        dst.at[c, pl.ds(NPT * s, NPT)])

    @pl.when(s == NS - 1)
    def _():
        pltpu.sync_copy(sh.at[pl.ds(NPT * (NS - 1), NPT_LAST)],
                        dst.at[c, pl.ds(NPT * (NS - 1), NPT_LAST)])


def _repack(idx1_v, idx2_v, n):
    for t in range(n // 16):
        idx2_v[t // 8, pl.ds((t % 8) * 16, 16)] = idx1_v[pl.ds(16 * t, 16)]


@functools.partial(
    pl.kernel,
    out_type=(
        jax.ShapeDtypeStruct((NC, N, IN), jnp.float32),
        jax.ShapeDtypeStruct((NC, N, IN), jnp.float32),
    ),
    mesh=_mesh,
    compiler_params=pltpu.CompilerParams(use_tc_tiling_on_sc=False, needs_layout_passes=False),
    scratch_types=[
        pltpu.VMEM((C_S, IN), jnp.float32),
        pltpu.VMEM((C_S, IN), jnp.float32),
        pltpu.VMEM((C_S,), jnp.int32),
        pltpu.VMEM((C_S,), jnp.int32),
        pltpu.VMEM((KS, 128), jnp.int32),
        pltpu.VMEM((KS, 128), jnp.int32),
        pltpu.VMEM_SHARED((N, IN), jnp.float32),
        pltpu.VMEM_SHARED((N, IN), jnp.float32),
    ],
)
def _k_scatter0(ea_hbm, n0_hbm, n1_hbm, sp_hbm, cp_hbm,
                ea_v, ones_v, n0l_v, n1l_v, n0_v, n1_v, s_sh, c_sh):
    c = lax.axis_index("c")
    s = lax.axis_index("s")
    wid = _wid()

    def fill(i, _):
        ea_v[i, :] = jnp.zeros((IN,), jnp.float32)
        ones_v[i, :] = jnp.ones((IN,), jnp.float32)
        return 0
    lax.fori_loop(0, C_S, fill, 0)

    _copy_slab(ea_v, s_sh, s)
    _copy_slab(ea_v, c_sh, s)
    plsc.subcore_barrier()

    def chunk(t, _):
        cid = wid + NW * t
        eb = pl.multiple_of(cid * C_S, C_S)
        pltpu.sync_copy(ea_hbm.at[pl.ds(eb, C_S)], ea_v)
        pltpu.sync_copy(n0_hbm.at[pl.ds(eb, C_S)], n0l_v)
        pltpu.sync_copy(n1_hbm.at[pl.ds(eb, C_S)], n1l_v)
        _repack(n0l_v, n0_v, C_S)
        _repack(n1l_v, n1_v, C_S)
        for j in range(KS):
            src = ea_v.at[pl.ds(128 * j, 128)]
            srco = ones_v.at[pl.ds(128 * j, 128)]
            pltpu.sync_copy(src, s_sh.at[n0_v.at[j]], add=True)
            pltpu.sync_copy(src, s_sh.at[n1_v.at[j]], add=True)
            pltpu.sync_copy(srco, c_sh.at[n0_v.at[j]], add=True)
            pltpu.sync_copy(srco, c_sh.at[n1_v.at[j]], add=True)
        return 0
    lax.fori_loop(0, _nchunks(wid, NCH_S), chunk, 0)

    plsc.subcore_barrier()
    _copy_slab_out(s_sh, sp_hbm, c, s)
    _copy_slab_out(c_sh, cp_hbm, c, s)


def _edge_body(layer1, xh_hbm, gv_hbm, wf_hbm, n0_hbm, n1_hbm, bias_hbm,
               bt_hbm, x1_hbm, sp_hbm, pp_hbm, xh_v, g0_v, g1_v, n0_v, n1_v,
               wf_v, b_v, rd_v, n0r_v, n1r_v, gid_v, bt_v, s_sh, p_sh, zb_v):
    c = lax.axis_index("c")
    s = lax.axis_index("s")
    wid = _wid()

    pltpu.sync_copy(wf_hbm, wf_v)
    pltpu.sync_copy(bias_hbm, b_v)
    pltpu.sync_copy(bt_hbm, bt_v)

    if layer1:
        _zero_rows(zb_v, 80, HID)

        @pl.when(s < NS - 1)
        def _():
            for r in range(NPT // 80):
                pltpu.sync_copy(zb_v, s_sh.at[pl.ds(NPT * s + 80 * r, 80)])

        @pl.when(s == NS - 1)
        def _():
            for r in range(NPT_LAST // 80):
                pltpu.sync_copy(
                    zb_v, s_sh.at[pl.ds(NPT * (NS - 1) + 80 * r, 80)])
        zsrc = zb_v
    else:
        _zero_rows(xh_v, G, HID)
        zsrc = xh_v

    @pl.when(s == 0)
    def _():
        pltpu.sync_copy(zsrc.at[pl.ds(0, G)], p_sh)
    plsc.subcore_barrier()

    b_ks = [b_v[pl.ds(16 * k, 16)] for k in range(HID // 16)]

    def chunk(t, _):
        cid = wid + NW * t
        eb = pl.multiple_of(cid * C_E, C_E)
        pltpu.sync_copy(n0_hbm.at[pl.ds(eb, C_E)], n0_v)
        pltpu.sync_copy(n1_hbm.at[pl.ds(eb, C_E)], n1_v)
        for j in range(KE):
            pltpu.sync_copy(gv_hbm.at[n0_v.at[pl.ds(128 * j, 128)]],
                            g0_v.at[pl.ds(128 * j, 128)])
            pltpu.sync_copy(gv_hbm.at[n1_v.at[pl.ds(128 * j, 128)]],
                            g1_v.at[pl.ds(128 * j, 128)])
        pltpu.sync_copy(xh_hbm.at[pl.ds(eb, C_E)], xh_v)
        for t2 in range(C_E // 16):
            nv0 = n0_v[pl.ds(16 * t2, 16)]
            nv1 = n1_v[pl.ds(16 * t2, 16)]
            w0 = plsc.load_gather(wf_v, [nv0 >> 7, nv0 & 127])
            w1 = plsc.load_gather(wf_v, [nv1 >> 7, nv1 & 127])
            rd_v[pl.ds(16 * t2, 16)] = 1.0 / (1.0 + w0 + w1)
            gid_v[t2 // 8, pl.ds((t2 % 8) * 16, 16)] = plsc.load_gather(
                bt_v, [nv0 >> 7, nv0 & 127])
            if layer1:
                n0r_v[t2 // 8, pl.ds((t2 % 8) * 16, 16)] = nv0
                n1r_v[t2 // 8, pl.ds((t2 % 8) * 16, 16)] = nv1

        def row(i, _):
            rs = plsc.load_gather(rd_v, [jnp.full((16,), i, jnp.int32)])
            for k in range(HID // 16):
                sl = pl.ds(16 * k, 16)
                v = xh_v[i, sl] + g0_v[i, sl] + g1_v[i, sl]
                xh_v[i, sl] = jnp.maximum(v * rs + b_ks[k], 0.0)
            return 0
        lax.fori_loop(0, C_E, row, 0)

        if layer1:
            pltpu.sync_copy(xh_v, x1_hbm.at[pl.ds(eb, C_E)])
            for j in range(KE):
                src = xh_v.at[pl.ds(128 * j, 128)]
                pltpu.sync_copy(src, s_sh.at[n0r_v.at[j]], add=True)
                pltpu.sync_copy(src, s_sh.at[n1r_v.at[j]], add=True)
                pltpu.sync_copy(src, p_sh.at[gid_v.at[j]], add=True)
        else:
            for j in range(KE):
                pltpu.sync_copy(xh_v.at[pl.ds(128 * j, 128)],
                                p_sh.at[gid_v.at[j]], add=True)
        return 0
    lax.fori_loop(0, _nchunks(wid, NCH_E), chunk, 0)

    plsc.subcore_barrier()
    if layer1:
        _copy_slab_out(s_sh, sp_hbm, c, s)

    @pl.when(s == 0)
    def _():
        pltpu.sync_copy(p_sh, pp_hbm.at[c])


_edge_scratch = [
    pltpu.VMEM((C_E, HID), jnp.float32),
    pltpu.VMEM((C_E, HID), jnp.float32),
    pltpu.VMEM((C_E, HID), jnp.float32),
    pltpu.VMEM((C_E,), jnp.int32),
    pltpu.VMEM((C_E,), jnp.int32),
    pltpu.VMEM((80, 128), jnp.float32),
    pltpu.VMEM((HID,), jnp.float32),
    pltpu.VMEM((C_E,), jnp.float32),
    pltpu.VMEM((KE, 128), jnp.int32),
    pltpu.VMEM((KE, 128), jnp.int32),
    pltpu.VMEM((KE, 128), jnp.int32),
    pltpu.VMEM((80, 128), jnp.int32),
]


@functools.partial(
    pl.kernel,
    out_type=(
        jax.ShapeDtypeStruct((E, HID), jnp.float32),
        jax.ShapeDtypeStruct((NC, N, HID), jnp.float32),
        jax.ShapeDtypeStruct((NC, G, HID), jnp.float32),
    ),
    mesh=_mesh,
    compiler_params=pltpu.CompilerParams(use_tc_tiling_on_sc=False, needs_layout_passes=False),
    scratch_types=_edge_scratch + [
        pltpu.VMEM_SHARED((N, HID), jnp.float32),
        pltpu.VMEM_SHARED((G, HID), jnp.float32),
        pltpu.VMEM((80, HID), jnp.float32),
    ],
)
def _k_edge0(xh, gv, wf, n0, n1, b, bt, x1, sp, pp, *scr):
    _edge_body(True, xh, gv, wf, n0, n1, b, bt, x1, sp, pp, *scr)


@functools.partial(
    pl.kernel,
    out_type=jax.ShapeDtypeStruct((NC, G, HID), jnp.float32),
    mesh=_mesh,
    compiler_params=pltpu.CompilerParams(use_tc_tiling_on_sc=False, needs_layout_passes=False),
    scratch_types=_edge_scratch + [
        pltpu.VMEM_SHARED((G, HID), jnp.float32),
    ],
)
def _k_edge1(xh, gv, wf, n0, n1, b, bt, pp, *scr):
    sc = list(scr)
    _edge_body(False, xh, gv, wf, n0, n1, b, bt, None, None, pp,
               *sc[:12], None, sc[12], None)


def _mm_body(x_ref, w_ref, o_ref):
    o_ref[...] = jnp.dot(x_ref[...], w_ref[...],
                         preferred_element_type=jnp.float32)


def _mm(x, w, bm=1600):
    m, k = x.shape
    n = w.shape[1]
    return pl.pallas_call(
        _mm_body,
        grid=(m // bm,),
        in_specs=[pl.BlockSpec((bm, k), lambda i: (i, 0)),
                  pl.BlockSpec((k, n), lambda i: (0, 0))],
        out_specs=pl.BlockSpec((bm, n), lambda i: (i, 0)),
        out_shape=jax.ShapeDtypeStruct((m, n), jnp.float32),
    )(x, w)


def _tab0_body(sp_ref, cp_ref, w_ref, gv_ref, wf_ref):
    s = sp_ref[0] + sp_ref[1]
    cnt = cp_ref[0, :, :] + cp_ref[1, :, :]
    g = jnp.dot(s, w_ref[...], preferred_element_type=jnp.float32)
    c0 = cnt[:, :1]
    mask = jnp.where((c0 != 1.0) & (c0 > 0.0), 1.0, 0.0)
    csafe = jnp.where(c0 < 2.0, 1.0, c0)
    gv_ref[...] = g * (mask / csafe)
    wf_ref[...] = jnp.where(cnt != 1.0, 1.0, 0.0)


def _tab1_body(sp_ref, cp_ref, w_ref, gv_ref):
    s = sp_ref[0] + sp_ref[1]
    cnt = cp_ref[0, :, :1] + cp_ref[1, :, :1]
    g = jnp.dot(s, w_ref[...], preferred_element_type=jnp.float32)
    mask = jnp.where((cnt != 1.0) & (cnt > 0.0), 1.0, 0.0)
    csafe = jnp.where(cnt < 2.0, 1.0, cnt)
    gv_ref[...] = g * (mask / csafe)


def _final_body(p1_ref, p2_ref, o_ref):
    o_ref[:, :HID] = p1_ref[0] + p1_ref[1]
    o_ref[:, HID:] = p2_ref[0] + p2_ref[1]


@jax.jit
def kernel(x, edge_attr, edge_index, batch, W0, b0, W1, b1):
    n0 = edge_index[0]
    n1 = edge_index[1]
    btp = jnp.pad(batch, (0, 80 * 128 - N)).reshape(80, 128)

    xh0 = _mm(edge_attr, W0)
    s0p, c0p = _k_scatter0(edge_attr, n0, n1)

    gv0, wf_raw = pl.pallas_call(
        _tab0_body,
        out_shape=(jax.ShapeDtypeStruct((N, HID), jnp.float32),
                   jax.ShapeDtypeStruct((N, IN), jnp.float32)),
    )(s0p, c0p, W0)
    wfp = jnp.pad(wf_raw[:, 0], (0, 80 * 128 - N)).reshape(80, 128)

    x1, s1p, pool1 = _k_edge0(xh0, gv0, wfp, n0, n1, b0, btp)

    xh1 = _mm(x1, W1)

    gv1 = pl.pallas_call(
        _tab1_body,
        out_shape=jax.ShapeDtypeStruct((N, HID), jnp.float32),
    )(s1p, c0p, W1)

    pool2 = _k_edge1(xh1, gv1, wfp, n0, n1, b1, btp)

    return pl.pallas_call(
        _final_body,
        out_shape=jax.ShapeDtypeStruct((G, 2 * HID), jnp.float32),
    )(pool1, pool2)
